# Initial kernel scaffold; baseline (speedup 1.0000x reference)
import jax
import jax.numpy as jnp
from jax.experimental import pallas as pl


def kernel(x, y, train_mask):
    raise NotImplementedError("write your pallas kernel here")


# speedup vs baseline: 2.3683x; 2.3683x over previous
import jax
import jax.numpy as jnp
from jax.experimental import pallas as pl

N = 100000
D = 128
DIM_OUT = 40
BLOCK = 2000


def _encode_block(x_ref, y_ref, m_ref, o_ref):
    o_ref[:, :D] = x_ref[...]
    cols = jax.lax.broadcasted_iota(jnp.int32, (BLOCK, DIM_OUT), 1)
    o_ref[:, D:] = jnp.where(cols == y_ref[...], m_ref[...], 0.0)


def kernel(x, y, train_mask):
    n = x.shape[0]
    m = train_mask.astype(x.dtype).reshape(n, 1)
    grid = (n // BLOCK,)
    return pl.pallas_call(
        _encode_block,
        grid=grid,
        in_specs=[
            pl.BlockSpec((BLOCK, D), lambda i: (i, 0)),
            pl.BlockSpec((BLOCK, 1), lambda i: (i, 0)),
            pl.BlockSpec((BLOCK, 1), lambda i: (i, 0)),
        ],
        out_specs=pl.BlockSpec((BLOCK, D + DIM_OUT), lambda i: (i, 0)),
        out_shape=jax.ShapeDtypeStruct((n, D + DIM_OUT), x.dtype),
    )(x, y, m)


# speedup vs baseline: 2.4610x; 1.0391x over previous
import jax
import jax.numpy as jnp
from jax.experimental import pallas as pl
from jax.experimental.pallas import tpu as pltpu

N = 100000
D = 128
DIM_OUT = 40
BLOCK = 10000


def _encode_block(x_ref, y_ref, m_ref, o_ref):
    o_ref[:, :D] = x_ref[...]
    cols = jax.lax.broadcasted_iota(jnp.int32, (BLOCK, DIM_OUT), 1)
    o_ref[:, D:] = jnp.where(cols == y_ref[...], m_ref[...], 0.0)


def kernel(x, y, train_mask):
    n = x.shape[0]
    m = train_mask.astype(x.dtype).reshape(n, 1)
    grid = (n // BLOCK,)
    return pl.pallas_call(
        _encode_block,
        grid=grid,
        in_specs=[
            pl.BlockSpec((BLOCK, D), lambda i: (i, 0)),
            pl.BlockSpec((BLOCK, 1), lambda i: (i, 0)),
            pl.BlockSpec((BLOCK, 1), lambda i: (i, 0)),
        ],
        out_specs=pl.BlockSpec((BLOCK, D + DIM_OUT), lambda i: (i, 0)),
        out_shape=jax.ShapeDtypeStruct((n, D + DIM_OUT), x.dtype),
        compiler_params=pltpu.CompilerParams(
            dimension_semantics=("arbitrary",),
        ),
    )(x, y, m)


# speedup vs baseline: 4.2386x; 1.7223x over previous
import jax
import jax.numpy as jnp
from jax.experimental import pallas as pl
from jax.experimental.pallas import tpu as pltpu

N = 100000
D = 128
DIM_OUT = 40
BLOCK = 10000


def _encode_block(x_ref, t_ref, o_ref):
    o_ref[:, :D] = x_ref[...]
    tcol = t_ref[0, 0, :].reshape(BLOCK, 1)
    cols = jax.lax.broadcasted_iota(jnp.int32, (BLOCK, DIM_OUT), 1)
    o_ref[:, D:] = (cols == tcol).astype(jnp.float32)


def kernel(x, y, train_mask):
    n = x.shape[0]
    grid = (n // BLOCK,)
    t = jnp.where(train_mask, y[:, 0], -1).reshape(grid[0], 1, BLOCK)
    return pl.pallas_call(
        _encode_block,
        grid=grid,
        in_specs=[
            pl.BlockSpec((BLOCK, D), lambda i: (i, 0)),
            pl.BlockSpec((1, 1, BLOCK), lambda i: (i, 0, 0)),
        ],
        out_specs=pl.BlockSpec((BLOCK, D + DIM_OUT), lambda i: (i, 0)),
        out_shape=jax.ShapeDtypeStruct((n, D + DIM_OUT), x.dtype),
        compiler_params=pltpu.CompilerParams(
            dimension_semantics=("arbitrary",),
        ),
    )(x, t)
